# baseline (device time: 15766 ns/iter reference)
import jax
import jax.numpy as jnp
from jax import lax
from jax.experimental import pallas as pl
from jax.experimental.pallas import tpu as pltpu

C = 8


def kernel(x):
    m, n = x.shape
    H = m // 2
    R = H // C

    def body(
        x_hbm,
        out_hbm,
        xv,
        obuf,
        ybuf,
        in_sems,
        osem,
        ldsems,
        ysend,
        yrecv,
        xsend,
        xrecv,
    ):
        my_x = lax.axis_index("x")
        my_y = lax.axis_index("y")
        ynbr = (my_x, 1 - my_y)
        xnbr = (1 - my_x, my_y)

        my_row0 = my_y * m
        wire0 = my_x * H
        yland0 = (1 - my_y) * m + my_x * H

        in_a = pltpu.make_async_copy(
            x_hbm.at[pl.ds(wire0, H)], xv.at[pl.ds(0, H)], in_sems.at[0]
        )
        in_a.start()
        oth0 = (1 - my_x) * H
        in_b = pltpu.make_async_copy(
            x_hbm.at[pl.ds(oth0, H)], xv.at[pl.ds(H, H)], in_sems.at[1]
        )
        in_b.start()

        barrier = pltpu.get_barrier_semaphore()
        for nbr in (ynbr, xnbr):
            pl.semaphore_signal(
                barrier, inc=1, device_id=nbr, device_id_type=pl.DeviceIdType.MESH
            )
        pl.semaphore_wait(barrier, 2)

        in_a.wait()
        yrdmas = []
        for c in range(C):
            osl = pl.ds(wire0 + c * R, R)
            obuf[osl, :] = xv[pl.ds(c * R, R), :].astype(jnp.bfloat16)
            r = pltpu.make_async_remote_copy(
                src_ref=obuf.at[osl],
                dst_ref=ybuf.at[pl.ds(c * R, R)],
                send_sem=ysend.at[c],
                recv_sem=yrecv.at[c],
                device_id=ynbr,
                device_id_type=pl.DeviceIdType.MESH,
            )
            r.start()
            yrdmas.append(r)

        in_b.wait()
        obuf[pl.ds(oth0, H), :] = xv[pl.ds(H, H), :].astype(jnp.bfloat16)
        own = pltpu.make_async_copy(obuf, out_hbm.at[pl.ds(my_row0, m)], osem)
        own.start()

        xrdmas = []
        lds = []
        for c in range(C):
            vsl = pl.ds(c * R, R)
            gsl = pl.ds(yland0 + c * R, R)
            yrdmas[c].wait_recv()
            r = pltpu.make_async_remote_copy(
                src_ref=ybuf.at[vsl],
                dst_ref=out_hbm.at[gsl],
                send_sem=xsend.at[c],
                recv_sem=xrecv.at[c],
                device_id=xnbr,
                device_id_type=pl.DeviceIdType.MESH,
            )
            r.start()
            xrdmas.append(r)
            ld = pltpu.make_async_copy(ybuf.at[vsl], out_hbm.at[gsl], ldsems.at[c])
            ld.start()
            lds.append(ld)

        for c in range(C):
            xrdmas[c].wait_recv()
        own.wait()
        for c in range(C):
            lds[c].wait()
            yrdmas[c].wait_send()
            xrdmas[c].wait_send()

    return pl.pallas_call(
        body,
        out_shape=jax.ShapeDtypeStruct((2 * m, n), jnp.bfloat16),
        in_specs=[pl.BlockSpec(memory_space=pl.ANY)],
        out_specs=pl.BlockSpec(memory_space=pl.ANY),
        scratch_shapes=[
            pltpu.VMEM((m, n), jnp.float32),
            pltpu.VMEM((m, n), jnp.bfloat16),
            pltpu.VMEM((H, n), jnp.bfloat16),
            pltpu.SemaphoreType.DMA((2,)),
            pltpu.SemaphoreType.DMA,
            pltpu.SemaphoreType.DMA((C,)),
            pltpu.SemaphoreType.DMA((C,)),
            pltpu.SemaphoreType.DMA((C,)),
            pltpu.SemaphoreType.DMA((C,)),
            pltpu.SemaphoreType.DMA((C,)),
        ],
        compiler_params=pltpu.CompilerParams(collective_id=0),
    )(x)
